# baseline (speedup 1.0000x reference)
import functools

import jax
import jax.numpy as jnp
from jax import lax
from jax.experimental import pallas as pl
from jax.experimental.pallas import tpu as pltpu
from jax.experimental.pallas import tpu_sc as plsc

TEMP = 0.1
EPS = 1e-12

B = 4096
D = 32
DP = 128
BM = 512
CHUNK = 128
VU = 7176
VI = 10728
BT = 2 * B


def _prep_body(ut_ref, it_ref, uid_ref, iid_ref, tab_ref, ids_ref):
    ut = ut_ref[...]
    un = jnp.sqrt(jnp.sum(ut * ut, axis=0, keepdims=True))
    tab_ref[:VU, :D] = (ut / jnp.maximum(un, EPS)).T
    it = it_ref[...]
    inorm = jnp.sqrt(jnp.sum(it * it, axis=0, keepdims=True))
    tab_ref[VU:, :D] = (it / jnp.maximum(inorm, EPS)).T
    ids_ref[:B] = uid_ref[...]
    ids_ref[B:] = iid_ref[...] + VU


def _prep(u_table, i_table, u_ids, i_ids):
    return pl.pallas_call(
        _prep_body,
        out_shape=[
            jax.ShapeDtypeStruct((VU + VI, DP), jnp.float32),
            jax.ShapeDtypeStruct((BT,), jnp.int32),
        ],
    )(u_table.T, i_table.T, u_ids, i_ids)


def _sc_gather(ids, tab):
    info = plsc.get_sparse_core_info()
    nc, ns = info.num_cores, info.num_subcores
    nw = nc * ns
    b_per_w = BT // nw
    nchunk = b_per_w // CHUNK

    mesh = plsc.VectorSubcoreMesh(core_axis_name="c", subcore_axis_name="s")

    @functools.partial(
        pl.kernel,
        mesh=mesh,
        compiler_params=pltpu.CompilerParams(use_tc_tiling_on_sc=False),
        out_type=jax.ShapeDtypeStruct((BT, DP), jnp.float32),
        scratch_types=[
            pltpu.VMEM((b_per_w,), jnp.int32),
            pltpu.VMEM((b_per_w, DP), jnp.float32),
            pltpu.SemaphoreType.DMA,
            pltpu.SemaphoreType.DMA,
        ],
    )
    def gather_k(ids_hbm, tab_hbm, emb_out, idx_v, rows_v, idsem, gsem):
        wid = lax.axis_index("s") * nc + lax.axis_index("c")
        base = wid * b_per_w
        pltpu.async_copy(ids_hbm.at[pl.ds(base, b_per_w)], idx_v,
                         idsem).wait()
        gathers = []
        for c in range(nchunk):
            gathers.append(pltpu.async_copy(
                tab_hbm.at[idx_v.at[pl.ds(c * CHUNK, CHUNK)]],
                rows_v.at[pl.ds(c * CHUNK, CHUNK)], gsem))
        for g in gathers:
            g.wait()
        pltpu.async_copy(rows_v, emb_out.at[pl.ds(base, b_per_w)],
                         gsem).wait()

    return gather_k(ids, tab)


def _tc_body(u_ref, i_ref, out_ref):
    u = u_ref[:, :D]
    i = i_ref[:, :D]
    out_ref[...] = lax.dot_general(
        u, i, (((1,), (1,)), ((), ())),
        preferred_element_type=jnp.float32,
    ) * (1.0 / TEMP)


def kernel(u_ids, i_ids, u_table, i_table):
    tab, ids = _prep(u_table, i_table,
                     u_ids.astype(jnp.int32), i_ids.astype(jnp.int32))
    emb = _sc_gather(ids, tab)

    return pl.pallas_call(
        _tc_body,
        grid=(B // BM,),
        in_specs=[
            pl.BlockSpec((BM, DP), lambda m: (m, 0)),
            pl.BlockSpec((B, DP), lambda m: (1, 0)),
        ],
        out_specs=pl.BlockSpec((BM, B), lambda m: (m, 0)),
        out_shape=jax.ShapeDtypeStruct((B, B), jnp.float32),
    )(emb, emb)
